# baseline (speedup 1.0000x reference)
import functools

import jax
import jax.numpy as jnp
from jax import lax
from jax.experimental import pallas as pl
from jax.experimental.pallas import tpu as pltpu
from jax.experimental.pallas import tpu_sc as plsc

_NW = 32


def _rb(v):
    return v.astype(jnp.bfloat16).astype(jnp.float32)


def _mm_body(x_ref, w_ref, b_ref, o_ref):
    o_ref[...] = jnp.dot(_rb(x_ref[...]), w_ref[...],
                         preferred_element_type=jnp.float32) + b_ref[...]


@functools.lru_cache(maxsize=None)
def _mm_call(R, Cp, Dp, BR):
    return pl.pallas_call(
        _mm_body,
        grid=(R // BR,),
        in_specs=[
            pl.BlockSpec((BR, Cp), lambda i: (i, 0)),
            pl.BlockSpec((Cp, Dp), lambda i: (0, 0)),
            pl.BlockSpec((1, Dp), lambda i: (0, 0)),
        ],
        out_specs=pl.BlockSpec((BR, Dp), lambda i: (i, 0)),
        out_shape=jax.ShapeDtypeStruct((R, Dp), jnp.float32),
    )


def _matmul(X, W, bias=None):
    R, C = X.shape
    D = W.shape[1]
    Cp = -(-C // 128) * 128
    Dp = -(-D // 128) * 128
    if Cp != C:
        X = jnp.pad(X, ((0, 0), (0, Cp - C)))
        W = jnp.pad(W, ((0, Cp - C), (0, 0)))
    if Dp != D:
        W = jnp.pad(W, ((0, 0), (0, Dp - D)))
    b = jnp.zeros((1, Dp), jnp.float32) if bias is None else jnp.pad(
        bias.reshape(1, D), ((0, 0), (0, Dp - D)))
    BR = 512 if R % 512 == 0 else (256 if R % 256 == 0 else R)
    out = _mm_call(R, Cp, Dp, BR)(X, W, b)
    return out[:, :D] if Dp != D else out


@functools.lru_cache(maxsize=None)
def _sc_gather_reduce_call(Rsrc, D, Q, k):
    assert Q % _NW == 0
    qpw = Q // _NW
    G = max(1, min(qpw, 128 // k))
    assert qpw % G == 0
    ngroups = qpw // G
    Gk = G * k
    nch = D // 16
    cpg = min(8, nch)
    ncg = nch // cpg
    mesh = plsc.VectorSubcoreMesh(core_axis_name="c", subcore_axis_name="s")

    @functools.partial(
        pl.kernel,
        mesh=mesh,
        out_type=[jax.ShapeDtypeStruct((Q, D), jnp.float32)] * 3,
        scratch_types=[
            pltpu.VMEM((Gk,), jnp.int32),
            pltpu.VMEM((Gk, D), jnp.float32),
            pltpu.VMEM((G, D), jnp.float32),
            pltpu.VMEM((G, D), jnp.float32),
            pltpu.VMEM((G, D), jnp.float32),
            pltpu.SemaphoreType.DMA,
        ],
    )
    def kern(a_hbm, idx_hbm, omax, osum, osq, idx_g, rows, mb, sb, qb2, sem):
        wid = lax.axis_index("s") * 2 + lax.axis_index("c")
        q0 = wid * qpw

        def gbody(gi, _):
            qb = q0 + gi * G
            pltpu.sync_copy(idx_hbm.at[pl.ds(qb * k, Gk)], idx_g)
            pltpu.async_copy(a_hbm.at[idx_g], rows, sem).wait()

            def qbody(q, _):
                for cg in range(ncg):
                    def rbody(r, acc):
                        row = q * k + r
                        out = []
                        for c in range(cpg):
                            v = rows[row, pl.ds((cg * cpg + c) * 16, 16)]
                            m, s, t = acc[3 * c], acc[3 * c + 1], acc[3 * c + 2]
                            out += [jnp.maximum(m, v), s + v, t + v * v]
                        return tuple(out)

                    init = []
                    for _c in range(cpg):
                        init += [jnp.full((16,), -1e30, jnp.float32),
                                 jnp.zeros((16,), jnp.float32),
                                 jnp.zeros((16,), jnp.float32)]
                    acc = lax.fori_loop(0, k, rbody, tuple(init))
                    for c in range(cpg):
                        sl = pl.ds((cg * cpg + c) * 16, 16)
                        mb[q, sl] = acc[3 * c]
                        sb[q, sl] = acc[3 * c + 1]
                        qb2[q, sl] = acc[3 * c + 2]
                return 0

            lax.fori_loop(0, G, qbody, 0)
            pltpu.sync_copy(mb, omax.at[pl.ds(qb, G)])
            pltpu.sync_copy(sb, osum.at[pl.ds(qb, G)])
            pltpu.sync_copy(qb2, osq.at[pl.ds(qb, G)])
            return 0

        lax.fori_loop(0, ngroups, gbody, 0)

    return kern


def _sc_gather_reduce(A, idx_flat, k):
    Rsrc, D = A.shape
    Q = idx_flat.shape[0] // k
    return _sc_gather_reduce_call(Rsrc, D, Q, k)(A, idx_flat)


@functools.lru_cache(maxsize=None)
def _sc_gather_xyz_call(B, N, k):
    E = B * N * k
    epw = E // _NW
    GRP = 128
    NB = 4
    CH = GRP * NB
    nch = epw // CH
    mesh = plsc.VectorSubcoreMesh(core_axis_name="c", subcore_axis_name="s")

    @functools.partial(
        pl.kernel,
        mesh=mesh,
        out_type=[jax.ShapeDtypeStruct((E,), jnp.float32)] * 3,
        scratch_types=[
            pltpu.VMEM((CH,), jnp.int32),
            pltpu.VMEM((NB, GRP), jnp.float32),
            pltpu.VMEM((NB, GRP), jnp.float32),
            pltpu.VMEM((NB, GRP), jnp.float32),
            pltpu.SemaphoreType.DMA,
        ],
    )
    def kern(xh, yh, zh, idxh, ox_h, oy_h, oz_h, iv, xr, yr, zr, sem):
        wid = lax.axis_index("s") * 2 + lax.axis_index("c")
        w0 = wid * epw

        def chunk(ci, _):
            base = w0 + ci * CH
            pltpu.sync_copy(idxh.at[pl.ds(base, CH)], iv)
            cps = []
            for j in range(NB):
                ij = iv.at[pl.ds(j * GRP, GRP)]
                cps.append(pltpu.async_copy(xh.at[ij], xr.at[j], sem))
                cps.append(pltpu.async_copy(yh.at[ij], yr.at[j], sem))
                cps.append(pltpu.async_copy(zh.at[ij], zr.at[j], sem))
            for c in cps:
                c.wait()
            for j in range(NB):
                sl = pl.ds(base + j * GRP, GRP)
                pltpu.sync_copy(xr.at[j], ox_h.at[sl])
                pltpu.sync_copy(yr.at[j], oy_h.at[sl])
                pltpu.sync_copy(zr.at[j], oz_h.at[sl])
            return 0

        lax.fori_loop(0, nch, chunk, 0)

    return kern


def _l1_body(nbx, nby, nbz, cref, w_ref, omx, os1, os2):
    cx = cref[:, 0][:, None]
    cy = cref[:, 1][:, None]
    cz = cref[:, 2][:, None]
    dx = nbx[...] - cx
    dy = nby[...] - cy
    dz = nbz[...] - cz
    dist = jnp.sqrt(dx * dx + dy * dy + dz * dz + 1e-12)
    inv = 1.0 / (dist + 1e-8)
    ux, uy, uz = dx * inv, dy * inv, dz * inv
    cn = jnp.sqrt(cx * cx + cy * cy + cz * cz + 1e-12) + 1e-8
    cnx, cny, cnz = cx / cn, cy / cn, cz / cn
    dot = cnx * ux + cny * uy + cnz * uz
    rcx, rcy, rcz = _rb(cx), _rb(cy), _rb(cz)
    base = (rcx * _rb(w_ref[0])[None, :] + rcy * _rb(w_ref[1])[None, :]
            + rcz * _rb(w_ref[2])[None, :])
    h = jnp.broadcast_to(base[:, None, :],
                         (base.shape[0], nbx.shape[1], base.shape[1]))
    for arr, c in ((nbx[...], 3), (nby[...], 4), (nbz[...], 5),
                   (dx, 6), (dy, 7), (dz, 8), (dist, 9),
                   (ux, 10), (uy, 11), (uz, 12), (dot, 13)):
        h = h + _rb(arr)[:, :, None] * _rb(w_ref[c])[None, None, :]
    omx[...] = jnp.max(h, axis=1)
    os1[...] = jnp.sum(h, axis=1)
    os2[...] = jnp.sum(h * h, axis=1)


@functools.lru_cache(maxsize=None)
def _l1_conv_call(R, K, D, BQ):
    grid = (R // BQ,)
    return pl.pallas_call(
        _l1_body,
        grid=grid,
        in_specs=[
            pl.BlockSpec((BQ, K), lambda i: (i, 0)),
            pl.BlockSpec((BQ, K), lambda i: (i, 0)),
            pl.BlockSpec((BQ, K), lambda i: (i, 0)),
            pl.BlockSpec((BQ, 3), lambda i: (i, 0)),
            pl.BlockSpec((14, D), lambda i: (0, 0)),
        ],
        out_specs=[pl.BlockSpec((BQ, D), lambda i: (i, 0))] * 3,
        out_shape=[jax.ShapeDtypeStruct((R, D), jnp.float32)] * 3,
    )


def _geconv1_fast(xyz, p, k):
    B, N, _ = xyz.shape
    idx = _knn(xyz, xyz, k)
    xyzf = xyz.reshape(B * N, 3)
    xf = xyzf[:, 0] + 0.0
    yf = xyzf[:, 1] + 0.0
    zf = xyzf[:, 2] + 0.0
    offs = (jnp.arange(B, dtype=jnp.int32) * N)[:, None, None]
    nbx, nby, nbz = _sc_gather_xyz_call(B, N, k)(
        xf, yf, zf, (idx + offs).reshape(-1))
    D = p['W'].shape[1]
    mx, s1, s2 = _l1_conv_call(B * N, k, D, 64)(
        nbx.reshape(B * N, k), nby.reshape(B * N, k), nbz.reshape(B * N, k),
        xyzf, p['W'])
    E = B * N * k
    mu = jnp.sum(s1, 0) / E
    var = jnp.sum(s2, 0) / E - mu * mu
    h = p['g'] * (mx - mu) * jax.lax.rsqrt(var + 1e-5) + p['b']
    return jax.nn.leaky_relu(h, 0.2).reshape(B, N, D)


def _knn(ref, query, k):
    d2 = (jnp.sum(query ** 2, -1)[:, :, None]
          - 2.0 * jnp.einsum('bmc,bnc->bmn', query, ref)
          + jnp.sum(ref ** 2, -1)[:, None, :])
    _, idx = jax.lax.top_k(-d2, k)
    return idx


def _gather(points, idx):
    return jax.vmap(lambda p, i: p[i])(points, idx)


def _fps(xyz, npoint):
    B, N, _ = xyz.shape

    def body(i, carry):
        cent, dist, far = carry
        cent = cent.at[:, i].set(far)
        c = jnp.take_along_axis(xyz, far[:, None, None], axis=1)
        d = jnp.sum((xyz - c) ** 2, axis=-1)
        dist = jnp.minimum(dist, d)
        far = jnp.argmax(dist, axis=-1).astype(jnp.int32)
        return cent, dist, far

    cent0 = jnp.zeros((B, npoint), dtype=jnp.int32)
    dist0 = jnp.full((B, N), 1e10, dtype=xyz.dtype)
    far0 = jnp.zeros((B,), dtype=jnp.int32)
    cent, _, _ = jax.lax.fori_loop(0, npoint, body, (cent0, dist0, far0))
    return cent


def _bn(h, g, b, axes):
    m = jnp.mean(h, axis=axes, keepdims=True)
    v = jnp.var(h, axis=axes, keepdims=True)
    return g * (h - m) * jax.lax.rsqrt(v + 1e-5) + b


def _geconv1(feat, xyz, p, k):
    idx = _knn(xyz, xyz, k)
    nbr = _gather(xyz, idx)
    c = xyz[:, :, None, :]
    diff = nbr - c
    dist = jnp.sqrt(jnp.sum(diff * diff, -1, keepdims=True) + 1e-12)
    unit = diff / (dist + 1e-8)
    cn = c / (jnp.sqrt(jnp.sum(c * c, -1, keepdims=True) + 1e-12) + 1e-8)
    dot = jnp.sum(jnp.broadcast_to(cn, unit.shape) * unit, -1, keepdims=True)
    e = jnp.concatenate(
        [jnp.broadcast_to(c, nbr.shape), nbr, diff, dist, unit, dot], -1)
    h = jnp.einsum('bmkc,cd->bmkd', e, p['W'])
    h = jax.nn.leaky_relu(_bn(h, p['g'], p['b'], (0, 1, 2)), 0.2)
    return jnp.max(h, axis=2)


def _geconv_fast(feat, xyz, p, k, npoint):
    B, N, C = feat.shape
    if npoint is not None:
        fidx = _fps(xyz, npoint)
        q_xyz = _gather(xyz, fidx)
        q_feat = _gather(feat, fidx)
    else:
        q_xyz, q_feat = xyz, feat
    M = q_xyz.shape[1]
    idx = _knn(xyz, q_xyz, k)

    W = p['W']
    D = W.shape[1]
    W1r, W2r = _rb(W[:C]), _rb(W[C:])
    A = _matmul(feat.reshape(B * N, C), W1r)
    Bq = _matmul(q_feat.reshape(B * M, C), W2r - W1r)
    A = A - jnp.mean(A, 0)
    Bq = Bq - jnp.mean(Bq, 0)

    offs = (jnp.arange(B, dtype=jnp.int32) * N)[:, None, None]
    idx_flat = (idx + offs).reshape(-1)
    mx, s1, s2 = _sc_gather_reduce(A, idx_flat, k)

    E = B * M * k
    sum_h = jnp.sum(s1, 0) + k * jnp.sum(Bq, 0)
    sum_h2 = jnp.sum(s2, 0) + 2.0 * jnp.sum(Bq * s1, 0) + k * jnp.sum(Bq * Bq, 0)
    mu = sum_h / E
    var = sum_h2 / E - mu * mu
    h = p['g'] * (mx + Bq - mu) * jax.lax.rsqrt(var + 1e-5) + p['b']
    x = jax.nn.leaky_relu(h, 0.2).reshape(B, M, D)
    return x, q_xyz


def _geconv_ref(feat, xyz, p, k, npoint):
    if npoint is not None:
        fidx = _fps(xyz, npoint)
        q_xyz = _gather(xyz, fidx)
        q_feat = _gather(feat, fidx)
    else:
        q_xyz, q_feat = xyz, feat
    idx = _knn(xyz, q_xyz, k)
    nf = _gather(feat, idx)
    cf = q_feat[:, :, None, :]
    e = jnp.concatenate([nf - cf, jnp.broadcast_to(cf, nf.shape)], -1)
    h = jnp.einsum('bmkc,cd->bmkd', e, p['W'])
    h = jax.nn.leaky_relu(_bn(h, p['g'], p['b'], (0, 1, 2)), 0.2)
    return jnp.max(h, axis=2), q_xyz


def _fp(xyz1, xyz2, pts1, pts2, p):
    d2 = (jnp.sum(xyz1 ** 2, -1)[:, :, None]
          - 2.0 * jnp.einsum('bnc,bsc->bns', xyz1, xyz2)
          + jnp.sum(xyz2 ** 2, -1)[:, None, :])
    negd, idx = jax.lax.top_k(-d2, 3)
    d = jnp.maximum(-negd, 0.0)
    recip = 1.0 / (d + 1e-8)
    w = recip / jnp.sum(recip, -1, keepdims=True)
    nbr = _gather(pts2, idx)
    interp = jnp.sum(nbr * w[..., None], axis=2)
    h = jnp.concatenate([pts1, interp], -1)
    h = jax.nn.relu(_bn(jnp.einsum('bnc,cd->bnd', h, p['W1']),
                        p['g1'], p['b1'], (0, 1)))
    h = jax.nn.relu(_bn(jnp.einsum('bnc,cd->bnd', h, p['W2']),
                        p['g2'], p['b2'], (0, 1)))
    return h


def kernel(x, cls_label, params):
    xyz0 = jnp.transpose(x, (0, 2, 1))
    x1 = _geconv1_fast(xyz0, params['gec1'], 64)
    xyz1 = xyz0
    x2, xyz2 = _geconv_fast(x1, xyz1, params['gec2'], 64, 512)
    x3, xyz3 = _geconv_fast(x2, xyz2, params['gec3'], 128, None)
    x4, xyz4 = _geconv_ref(x3, xyz3, params['gec4'], 64, 64)
    x5, xyz5 = _geconv_ref(x4, xyz4, params['gec5'], 8, None)

    B, n5, _ = x5.shape
    g5 = jax.nn.leaky_relu(
        _bn(jnp.einsum('bnc,cd->bnd', x5, params['gconv']['W']),
            params['gconv']['g'], params['gconv']['b'], (0, 1)), 0.2)
    gl = jnp.concatenate([jnp.max(g5, axis=1), jnp.mean(g5, axis=1)], -1)
    x5c = jnp.concatenate(
        [g5, jnp.broadcast_to(gl[:, None, :], (B, n5, gl.shape[-1]))], -1)

    f3 = _fp(xyz3, xyz5, x3, x5c, params['fp5'])
    f1 = _fp(xyz1, xyz3, x1, f3, params['fp1'])

    N = f1.shape[1]
    cls = jnp.broadcast_to(cls_label[:, None, :],
                           (cls_label.shape[0], N, cls_label.shape[-1]))
    h = jnp.concatenate([f1, cls], -1)
    h = jax.nn.relu(_bn(jnp.einsum('bnc,cd->bnd', h, params['c1']['W'])
                        + params['c1']['bias'],
                        params['c1']['g'], params['c1']['b'], (0, 1)))
    h = jnp.einsum('bnc,cd->bnd', h, params['c2']['W']) + params['c2']['bias']
    return jax.nn.log_softmax(h, axis=-1)


# speedup vs baseline: 1.0010x; 1.0010x over previous
import functools

import jax
import jax.numpy as jnp
from jax import lax
from jax.experimental import pallas as pl
from jax.experimental.pallas import tpu as pltpu
from jax.experimental.pallas import tpu_sc as plsc

_NW = 32


def _rb(v):
    return v.astype(jnp.bfloat16).astype(jnp.float32)


def _mm_body(x_ref, w_ref, b_ref, o_ref):
    o_ref[...] = jnp.dot(_rb(x_ref[...]), w_ref[...],
                         preferred_element_type=jnp.float32) + b_ref[...]


@functools.lru_cache(maxsize=None)
def _mm_call(R, Cp, Dp, BR):
    return pl.pallas_call(
        _mm_body,
        grid=(R // BR,),
        in_specs=[
            pl.BlockSpec((BR, Cp), lambda i: (i, 0)),
            pl.BlockSpec((Cp, Dp), lambda i: (0, 0)),
            pl.BlockSpec((1, Dp), lambda i: (0, 0)),
        ],
        out_specs=pl.BlockSpec((BR, Dp), lambda i: (i, 0)),
        out_shape=jax.ShapeDtypeStruct((R, Dp), jnp.float32),
    )


def _matmul(X, W, bias=None):
    R, C = X.shape
    D = W.shape[1]
    Cp = -(-C // 128) * 128
    Dp = -(-D // 128) * 128
    if Cp != C:
        X = jnp.pad(X, ((0, 0), (0, Cp - C)))
        W = jnp.pad(W, ((0, Cp - C), (0, 0)))
    if Dp != D:
        W = jnp.pad(W, ((0, 0), (0, Dp - D)))
    b = jnp.zeros((1, Dp), jnp.float32) if bias is None else jnp.pad(
        bias.reshape(1, D), ((0, 0), (0, Dp - D)))
    BR = 512 if R % 512 == 0 else (256 if R % 256 == 0 else R)
    out = _mm_call(R, Cp, Dp, BR)(X, W, b)
    return out[:, :D] if Dp != D else out


@functools.lru_cache(maxsize=None)
def _sc_gather_reduce_call(Rsrc, D, Q, k):
    assert Q % _NW == 0
    qpw = Q // _NW
    G = max(1, min(qpw, 128 // k))
    assert qpw % G == 0
    ngroups = qpw // G
    Gk = G * k
    nch = D // 16
    cpg = min(8, nch)
    ncg = nch // cpg
    mesh = plsc.VectorSubcoreMesh(core_axis_name="c", subcore_axis_name="s")

    @functools.partial(
        pl.kernel,
        mesh=mesh,
        out_type=[jax.ShapeDtypeStruct((Q, D), jnp.float32)] * 3,
        scratch_types=[
            pltpu.VMEM((Gk,), jnp.int32),
            pltpu.VMEM((Gk, D), jnp.float32),
            pltpu.VMEM((G, D), jnp.float32),
            pltpu.VMEM((G, D), jnp.float32),
            pltpu.VMEM((G, D), jnp.float32),
            pltpu.SemaphoreType.DMA,
        ],
    )
    def kern(a_hbm, idx_hbm, omax, osum, osq, idx_g, rows, mb, sb, qb2, sem):
        wid = lax.axis_index("s") * 2 + lax.axis_index("c")
        q0 = wid * qpw

        def gbody(gi, _):
            qb = q0 + gi * G
            pltpu.sync_copy(idx_hbm.at[pl.ds(qb * k, Gk)], idx_g)
            pltpu.async_copy(a_hbm.at[idx_g], rows, sem).wait()

            def qbody(q, _):
                for cg in range(ncg):
                    def rbody(r, acc):
                        row = q * k + r
                        out = []
                        for c in range(cpg):
                            v = rows[row, pl.ds((cg * cpg + c) * 16, 16)]
                            m, s, t = acc[3 * c], acc[3 * c + 1], acc[3 * c + 2]
                            out += [jnp.maximum(m, v), s + v, t + v * v]
                        return tuple(out)

                    init = []
                    for _c in range(cpg):
                        init += [jnp.full((16,), -1e30, jnp.float32),
                                 jnp.zeros((16,), jnp.float32),
                                 jnp.zeros((16,), jnp.float32)]
                    acc = lax.fori_loop(0, k, rbody, tuple(init))
                    for c in range(cpg):
                        sl = pl.ds((cg * cpg + c) * 16, 16)
                        mb[q, sl] = acc[3 * c]
                        sb[q, sl] = acc[3 * c + 1]
                        qb2[q, sl] = acc[3 * c + 2]
                return 0

            lax.fori_loop(0, G, qbody, 0)
            pltpu.sync_copy(mb, omax.at[pl.ds(qb, G)])
            pltpu.sync_copy(sb, osum.at[pl.ds(qb, G)])
            pltpu.sync_copy(qb2, osq.at[pl.ds(qb, G)])
            return 0

        lax.fori_loop(0, ngroups, gbody, 0)

    return kern


def _sc_gather_reduce(A, idx_flat, k):
    Rsrc, D = A.shape
    Q = idx_flat.shape[0] // k
    return _sc_gather_reduce_call(Rsrc, D, Q, k)(A, idx_flat)


@functools.lru_cache(maxsize=None)
def _sc_gather_xyz_call(B, N, k):
    E = B * N * k
    epw = E // _NW
    GRP = 128
    NB = 4
    CH = GRP * NB
    nch = epw // CH
    mesh = plsc.VectorSubcoreMesh(core_axis_name="c", subcore_axis_name="s")

    @functools.partial(
        pl.kernel,
        mesh=mesh,
        out_type=[jax.ShapeDtypeStruct((E,), jnp.float32)] * 3,
        scratch_types=[
            pltpu.VMEM((CH,), jnp.int32),
            pltpu.VMEM((NB, GRP), jnp.float32),
            pltpu.VMEM((NB, GRP), jnp.float32),
            pltpu.VMEM((NB, GRP), jnp.float32),
            pltpu.SemaphoreType.DMA,
        ],
    )
    def kern(xh, yh, zh, idxh, ox_h, oy_h, oz_h, iv, xr, yr, zr, sem):
        wid = lax.axis_index("s") * 2 + lax.axis_index("c")
        w0 = wid * epw

        def chunk(ci, _):
            base = w0 + ci * CH
            pltpu.sync_copy(idxh.at[pl.ds(base, CH)], iv)
            cps = []
            for j in range(NB):
                ij = iv.at[pl.ds(j * GRP, GRP)]
                cps.append(pltpu.async_copy(xh.at[ij], xr.at[j], sem))
                cps.append(pltpu.async_copy(yh.at[ij], yr.at[j], sem))
                cps.append(pltpu.async_copy(zh.at[ij], zr.at[j], sem))
            for c in cps:
                c.wait()
            for j in range(NB):
                sl = pl.ds(base + j * GRP, GRP)
                pltpu.sync_copy(xr.at[j], ox_h.at[sl])
                pltpu.sync_copy(yr.at[j], oy_h.at[sl])
                pltpu.sync_copy(zr.at[j], oz_h.at[sl])
            return 0

        lax.fori_loop(0, nch, chunk, 0)

    return kern


def _l1_body(nbx, nby, nbz, cref, w_ref, omx, os1, os2):
    cx = cref[:, 0][:, None]
    cy = cref[:, 1][:, None]
    cz = cref[:, 2][:, None]
    dx = nbx[...] - cx
    dy = nby[...] - cy
    dz = nbz[...] - cz
    dist = jnp.sqrt(dx * dx + dy * dy + dz * dz + 1e-12)
    inv = 1.0 / (dist + 1e-8)
    ux, uy, uz = dx * inv, dy * inv, dz * inv
    cn = jnp.sqrt(cx * cx + cy * cy + cz * cz + 1e-12) + 1e-8
    cnx, cny, cnz = cx / cn, cy / cn, cz / cn
    dot = cnx * ux + cny * uy + cnz * uz
    rcx, rcy, rcz = _rb(cx), _rb(cy), _rb(cz)
    base = (rcx * _rb(w_ref[0])[None, :] + rcy * _rb(w_ref[1])[None, :]
            + rcz * _rb(w_ref[2])[None, :])
    h = jnp.broadcast_to(base[:, None, :],
                         (base.shape[0], nbx.shape[1], base.shape[1]))
    for arr, c in ((nbx[...], 3), (nby[...], 4), (nbz[...], 5),
                   (dx, 6), (dy, 7), (dz, 8), (dist, 9),
                   (ux, 10), (uy, 11), (uz, 12), (dot, 13)):
        h = h + _rb(arr)[:, :, None] * _rb(w_ref[c])[None, None, :]
    omx[...] = jnp.max(h, axis=1)
    os1[...] = jnp.sum(h, axis=1)
    os2[...] = jnp.sum(h * h, axis=1)


@functools.lru_cache(maxsize=None)
def _l1_conv_call(R, K, D, BQ):
    grid = (R // BQ,)
    return pl.pallas_call(
        _l1_body,
        grid=grid,
        in_specs=[
            pl.BlockSpec((BQ, K), lambda i: (i, 0)),
            pl.BlockSpec((BQ, K), lambda i: (i, 0)),
            pl.BlockSpec((BQ, K), lambda i: (i, 0)),
            pl.BlockSpec((BQ, 3), lambda i: (i, 0)),
            pl.BlockSpec((14, D), lambda i: (0, 0)),
        ],
        out_specs=[pl.BlockSpec((BQ, D), lambda i: (i, 0))] * 3,
        out_shape=[jax.ShapeDtypeStruct((R, D), jnp.float32)] * 3,
    )


def _geconv1_fast(xyz, p, k):
    B, N, _ = xyz.shape
    idx = _knn(xyz, xyz, k)
    xyzf = xyz.reshape(B * N, 3)
    xf = xyzf[:, 0] + 0.0
    yf = xyzf[:, 1] + 0.0
    zf = xyzf[:, 2] + 0.0
    offs = (jnp.arange(B, dtype=jnp.int32) * N)[:, None, None]
    nbx, nby, nbz = _sc_gather_xyz_call(B, N, k)(
        xf, yf, zf, (idx + offs).reshape(-1))
    D = p['W'].shape[1]
    mx, s1, s2 = _l1_conv_call(B * N, k, D, 64)(
        nbx.reshape(B * N, k), nby.reshape(B * N, k), nbz.reshape(B * N, k),
        xyzf, p['W'])
    E = B * N * k
    mu = jnp.sum(s1, 0) / E
    var = jnp.sum(s2, 0) / E - mu * mu
    h = p['g'] * (mx - mu) * jax.lax.rsqrt(var + 1e-5) + p['b']
    return jax.nn.leaky_relu(h, 0.2).reshape(B, N, D)


def _knn(ref, query, k):
    d2 = (jnp.sum(query ** 2, -1)[:, :, None]
          - 2.0 * jnp.einsum('bmc,bnc->bmn', query, ref)
          + jnp.sum(ref ** 2, -1)[:, None, :])
    _, idx = jax.lax.top_k(-d2, k)
    return idx


def _gather(points, idx):
    return jax.vmap(lambda p, i: p[i])(points, idx)


def _fps(xyz, npoint):
    B, N, _ = xyz.shape

    def body(i, carry):
        cent, dist, far = carry
        cent = cent.at[:, i].set(far)
        c = jnp.take_along_axis(xyz, far[:, None, None], axis=1)
        d = jnp.sum((xyz - c) ** 2, axis=-1)
        dist = jnp.minimum(dist, d)
        far = jnp.argmax(dist, axis=-1).astype(jnp.int32)
        return cent, dist, far

    cent0 = jnp.zeros((B, npoint), dtype=jnp.int32)
    dist0 = jnp.full((B, N), 1e10, dtype=xyz.dtype)
    far0 = jnp.zeros((B,), dtype=jnp.int32)
    cent, _, _ = jax.lax.fori_loop(0, npoint, body, (cent0, dist0, far0))
    return cent


def _bn(h, g, b, axes):
    m = jnp.mean(h, axis=axes, keepdims=True)
    v = jnp.var(h, axis=axes, keepdims=True)
    return g * (h - m) * jax.lax.rsqrt(v + 1e-5) + b


def _geconv_fast(feat, xyz, p, k, npoint):
    B, N, C = feat.shape
    if npoint is not None:
        fidx = _fps(xyz, npoint)
        q_xyz = _gather(xyz, fidx)
        q_feat = _gather(feat, fidx)
    else:
        q_xyz, q_feat = xyz, feat
    M = q_xyz.shape[1]
    idx = _knn(xyz, q_xyz, k)

    W = p['W']
    D = W.shape[1]
    W1r, W2r = _rb(W[:C]), _rb(W[C:])
    A = _matmul(feat.reshape(B * N, C), W1r)
    Bq = _matmul(q_feat.reshape(B * M, C), W2r - W1r)
    A = A - jnp.mean(A, 0)
    Bq = Bq - jnp.mean(Bq, 0)

    offs = (jnp.arange(B, dtype=jnp.int32) * N)[:, None, None]
    idx_flat = (idx + offs).reshape(-1)
    mx, s1, s2 = _sc_gather_reduce(A, idx_flat, k)

    E = B * M * k
    sum_h = jnp.sum(s1, 0) + k * jnp.sum(Bq, 0)
    sum_h2 = jnp.sum(s2, 0) + 2.0 * jnp.sum(Bq * s1, 0) + k * jnp.sum(Bq * Bq, 0)
    mu = sum_h / E
    var = sum_h2 / E - mu * mu
    h = p['g'] * (mx + Bq - mu) * jax.lax.rsqrt(var + 1e-5) + p['b']
    x = jax.nn.leaky_relu(h, 0.2).reshape(B, M, D)
    return x, q_xyz


def _geconv_ref(feat, xyz, p, k, npoint):
    if npoint is not None:
        fidx = _fps(xyz, npoint)
        q_xyz = _gather(xyz, fidx)
        q_feat = _gather(feat, fidx)
    else:
        q_xyz, q_feat = xyz, feat
    idx = _knn(xyz, q_xyz, k)
    nf = _gather(feat, idx)
    cf = q_feat[:, :, None, :]
    e = jnp.concatenate([nf - cf, jnp.broadcast_to(cf, nf.shape)], -1)
    h = jnp.einsum('bmkc,cd->bmkd', e, p['W'])
    h = jax.nn.leaky_relu(_bn(h, p['g'], p['b'], (0, 1, 2)), 0.2)
    return jnp.max(h, axis=2), q_xyz


def _fp(xyz1, xyz2, pts1, pts2, p):
    d2 = (jnp.sum(xyz1 ** 2, -1)[:, :, None]
          - 2.0 * jnp.einsum('bnc,bsc->bns', xyz1, xyz2)
          + jnp.sum(xyz2 ** 2, -1)[:, None, :])
    negd, idx = jax.lax.top_k(-d2, 3)
    d = jnp.maximum(-negd, 0.0)
    recip = 1.0 / (d + 1e-8)
    w = recip / jnp.sum(recip, -1, keepdims=True)
    nbr = _gather(pts2, idx)
    interp = jnp.sum(nbr * w[..., None], axis=2)
    h = jnp.concatenate([pts1, interp], -1)
    h = jax.nn.relu(_bn(jnp.einsum('bnc,cd->bnd', h, p['W1']),
                        p['g1'], p['b1'], (0, 1)))
    h = jax.nn.relu(_bn(jnp.einsum('bnc,cd->bnd', h, p['W2']),
                        p['g2'], p['b2'], (0, 1)))
    return h


def kernel(x, cls_label, params):
    xyz0 = jnp.transpose(x, (0, 2, 1))
    x1 = _geconv1_fast(xyz0, params['gec1'], 64)
    xyz1 = xyz0
    x2, xyz2 = _geconv_fast(x1, xyz1, params['gec2'], 64, 512)
    x3, xyz3 = _geconv_fast(x2, xyz2, params['gec3'], 128, None)
    x4, xyz4 = _geconv_ref(x3, xyz3, params['gec4'], 64, 64)
    x5, xyz5 = _geconv_ref(x4, xyz4, params['gec5'], 8, None)

    B, n5, _ = x5.shape
    g5 = jax.nn.leaky_relu(
        _bn(jnp.einsum('bnc,cd->bnd', x5, params['gconv']['W']),
            params['gconv']['g'], params['gconv']['b'], (0, 1)), 0.2)
    gl = jnp.concatenate([jnp.max(g5, axis=1), jnp.mean(g5, axis=1)], -1)
    x5c = jnp.concatenate(
        [g5, jnp.broadcast_to(gl[:, None, :], (B, n5, gl.shape[-1]))], -1)

    f3 = _fp(xyz3, xyz5, x3, x5c, params['fp5'])
    f1 = _fp(xyz1, xyz3, x1, f3, params['fp1'])

    N = f1.shape[1]
    cls = jnp.broadcast_to(cls_label[:, None, :],
                           (cls_label.shape[0], N, cls_label.shape[-1]))
    h = jnp.concatenate([f1, cls], -1)
    h = jax.nn.relu(_bn(jnp.einsum('bnc,cd->bnd', h, params['c1']['W'])
                        + params['c1']['bias'],
                        params['c1']['g'], params['c1']['b'], (0, 1)))
    h = jnp.einsum('bnc,cd->bnd', h, params['c2']['W']) + params['c2']['bias']
    return jax.nn.log_softmax(h, axis=-1)
